# baseline (speedup 1.0000x reference)
import jax
import jax.numpy as jnp
from jax.experimental import pallas as pl
from jax.experimental.pallas import tpu as pltpu

HIDDEN_DIM = 768
NUM_EXPERTS = 8
TOP_K = 2
N_TOKENS = 32768

BLOCK = 4096
N_STEPS = N_TOKENS // BLOCK
N_SPLIT = 4
SUB = BLOCK // N_SPLIT


def _sub_copy(x_hbm, xbuf, xsem, step, slot, q):
    return pltpu.make_async_copy(
        x_hbm.at[pl.ds(step * BLOCK + q * SUB, SUB)],
        xbuf.at[slot, pl.ds(q * SUB, SUB)],
        xsem.at[slot, q])


def _top2(scores, iota):
    m1 = jnp.max(scores, axis=0, keepdims=True)
    i1 = jnp.min(jnp.where(scores == m1, iota, NUM_EXPERTS),
                 axis=0, keepdims=True)
    masked = jnp.where(iota == i1, -1.0, scores)
    m2 = jnp.max(masked, axis=0, keepdims=True)
    i2 = jnp.min(jnp.where(masked == m2, iota, NUM_EXPERTS),
                 axis=0, keepdims=True)
    return jnp.concatenate([i1, i2], axis=0)


def _router_kernel(x_hbm, hp_hbm, idxt_ref, xbuf, hpbuf, xsem, hpsem):
    i = pl.program_id(0)

    @pl.when(i == 0)
    def _():
        for q in range(N_SPLIT):
            _sub_copy(x_hbm, xbuf, xsem, 0, 0, q).start()
        hp_cp = pltpu.make_async_copy(hp_hbm, hpbuf, hpsem)
        hp_cp.start()
        hp_cp.wait()

    @pl.when(i + 1 < N_STEPS)
    def _():
        for q in range(N_SPLIT):
            _sub_copy(x_hbm, xbuf, xsem, i + 1, (i + 1) % 2, q).start()

    hp = hpbuf[...]
    iota = jax.lax.broadcasted_iota(jnp.int32, (NUM_EXPERTS, SUB), 0)
    slot = i % 2
    for q in range(N_SPLIT):
        _sub_copy(x_hbm, xbuf, xsem, i, slot, q).wait()
        xq = xbuf[slot, pl.ds(q * SUB, SUB)]
        scores = jnp.abs(
            jax.lax.dot_general(
                hp, xq, (((1,), (1,)), ((), ())),
                preferred_element_type=jnp.float32,
            )
        )
        idxt_ref[:, pl.ds(q * SUB, SUB)] = _top2(scores, iota)


def kernel(x, hash_planes):
    n = x.shape[0]
    idxt = pl.pallas_call(
        _router_kernel,
        grid=(N_STEPS,),
        in_specs=[
            pl.BlockSpec(memory_space=pltpu.MemorySpace.HBM),
            pl.BlockSpec(memory_space=pltpu.MemorySpace.HBM),
        ],
        out_specs=pl.BlockSpec((TOP_K, BLOCK), lambda i: (0, i)),
        out_shape=jax.ShapeDtypeStruct((TOP_K, n), jnp.int32),
        scratch_shapes=[
            pltpu.VMEM((2, BLOCK, HIDDEN_DIM), jnp.float32),
            pltpu.VMEM((NUM_EXPERTS, HIDDEN_DIM), jnp.float32),
            pltpu.SemaphoreType.DMA((2, N_SPLIT)),
            pltpu.SemaphoreType.DMA,
        ],
        compiler_params=pltpu.CompilerParams(
            dimension_semantics=("arbitrary",),
        ),
    )(x, hash_planes)
    topk_indices = idxt.T
    topk_probs = jnp.full((n, TOP_K), 1.0 / TOP_K, jnp.float32)
    probs_uniform = jnp.full((n, NUM_EXPERTS), 1.0 / NUM_EXPERTS, jnp.float32)
    return (topk_indices, topk_probs, probs_uniform)
